# Initial kernel scaffold; baseline (speedup 1.0000x reference)
import jax
import jax.numpy as jnp
from jax.experimental import pallas as pl


def kernel(x, edge_index, W_l0, b_l0, W_r0, W_l1, b_l1, W_r1):
    raise NotImplementedError("write your pallas kernel here")


# speedup vs baseline: 7.3180x; 7.3180x over previous
import functools

import jax
import jax.numpy as jnp
from jax import lax
from jax.experimental import pallas as pl
from jax.experimental.pallas import tpu as pltpu
from jax.experimental.pallas import tpu_sc as plsc

_NC = 2
_NS = 16
_NW = _NC * _NS


def _pick_chunk(e_per_tile):
    for c in range(128, 0, -8):
        if e_per_tile % c == 0:
            return c
    raise ValueError(f"no valid chunk size for {e_per_tile} edges per tile")


@functools.lru_cache(maxsize=None)
def _make_sc_agg(n, d, e):
    e_per_tile = e // _NW
    c = _pick_chunk(e_per_tile)
    n_chunks = e_per_tile // c
    rows_per_tile = n // _NS
    mesh = plsc.VectorSubcoreMesh(
        core_axis_name="c", subcore_axis_name="s",
        num_cores=_NC, num_subcores=_NS)

    @functools.partial(
        pl.kernel,
        mesh=mesh,
        compiler_params=pltpu.CompilerParams(use_tc_tiling_on_sc=False),
        out_type=jax.ShapeDtypeStruct((_NC, n, d), jnp.float32),
        scratch_types=[
            pltpu.VMEM((n_chunks, c), jnp.int32),
            pltpu.VMEM((n_chunks, c), jnp.int32),
            pltpu.VMEM((c, d), jnp.float32),
            pltpu.VMEM_SHARED((n, d), jnp.float32),
            pltpu.SemaphoreType.DMA,
        ],
    )
    def agg(x_hbm, src_hbm, dst_hbm, zero_hbm, out_hbm,
            src_v, dst_v, rows_v, acc_sh, sem):
        cid = lax.axis_index("c")
        sid = lax.axis_index("s")
        wid = sid * _NC + cid
        row0 = sid * rows_per_tile
        pltpu.sync_copy(zero_hbm, acc_sh.at[pl.ds(row0, rows_per_tile)])
        pltpu.sync_copy(src_hbm.at[wid], src_v)
        pltpu.sync_copy(dst_hbm.at[wid], dst_v)
        plsc.subcore_barrier()

        def body(j, carry):
            pltpu.async_copy(x_hbm.at[src_v.at[j]], rows_v, sem).wait()
            pltpu.sync_copy(rows_v, acc_sh.at[dst_v.at[j]], add=True)
            return carry

        lax.fori_loop(0, n_chunks, body, 0)
        plsc.subcore_barrier()
        pltpu.sync_copy(acc_sh.at[pl.ds(row0, rows_per_tile)],
                        out_hbm.at[cid, pl.ds(row0, rows_per_tile)])

    return agg


def _tc1_body(part_ref, x_ref, wl_ref, b_ref, wr_ref, h_ref, deg_ref, d_in):
    p0 = part_ref[0]
    p1 = part_ref[1]
    agg = p0[:, :d_in] + p1[:, :d_in]
    deg = jnp.maximum(p0[:, d_in] + p1[:, d_in], 1.0)
    mean = agg / deg[:, None]
    acc = lax.dot_general(mean, wl_ref[...], (((1,), (1,)), ((), ())),
                          preferred_element_type=jnp.float32)
    acc = acc + lax.dot_general(x_ref[...], wr_ref[...], (((1,), (1,)), ((), ())),
                                preferred_element_type=jnp.float32)
    acc = acc + b_ref[...]
    h_ref[...] = jnp.maximum(acc, 0.0)
    deg_ref[...] = jnp.broadcast_to(deg[:, None], deg_ref.shape)


def _tc2_body(part_ref, deg_ref, h_ref, wl_ref, b_ref, wr_ref, out_ref):
    agg = part_ref[0] + part_ref[1]
    mean = agg / deg_ref[...]
    acc = lax.dot_general(mean, wl_ref[...], (((1,), (1,)), ((), ())),
                          preferred_element_type=jnp.float32)
    acc = acc + lax.dot_general(h_ref[...], wr_ref[...], (((1,), (1,)), ((), ())),
                                preferred_element_type=jnp.float32)
    out_ref[...] = acc + b_ref[...]


def kernel(x, edge_index, W_l0, b_l0, W_r0, W_l1, b_l1, W_r1):
    n, d_in = x.shape
    e = edge_index.shape[1]
    h0 = W_l0.shape[0]
    h1 = W_l1.shape[0]
    pad = 16
    dp = d_in + pad

    e_per_tile = e // _NW
    c = _pick_chunk(e_per_tile)
    n_chunks = e_per_tile // c
    srcs = edge_index[0].reshape(_NW, n_chunks, c)
    dsts = edge_index[1].reshape(_NW, n_chunks, c)

    x_aug = jnp.concatenate([x, jnp.ones((n, pad), jnp.float32)], axis=1)
    zeros_dp = jnp.zeros((n // _NS, dp), jnp.float32)
    zeros_h0 = jnp.zeros((n // _NS, h0), jnp.float32)

    part1 = _make_sc_agg(n, dp, e)(x_aug, srcs, dsts, zeros_dp)

    bn = 1000
    grid = (n // bn,)
    full = lambda i: (0, 0)
    h, degb = pl.pallas_call(
        functools.partial(_tc1_body, d_in=d_in),
        grid=grid,
        in_specs=[
            pl.BlockSpec((2, bn, dp), lambda i: (0, i, 0)),
            pl.BlockSpec((bn, d_in), lambda i: (i, 0)),
            pl.BlockSpec((h0, d_in), full),
            pl.BlockSpec((1, h0), full),
            pl.BlockSpec((h0, d_in), full),
        ],
        out_specs=[
            pl.BlockSpec((bn, h0), lambda i: (i, 0)),
            pl.BlockSpec((bn, h0), lambda i: (i, 0)),
        ],
        out_shape=[
            jax.ShapeDtypeStruct((n, h0), jnp.float32),
            jax.ShapeDtypeStruct((n, h0), jnp.float32),
        ],
    )(part1, x, W_l0, b_l0.reshape(1, h0), W_r0)

    part2 = _make_sc_agg(n, h0, e)(h, srcs, dsts, zeros_h0)

    out = pl.pallas_call(
        _tc2_body,
        grid=grid,
        in_specs=[
            pl.BlockSpec((2, bn, h0), lambda i: (0, i, 0)),
            pl.BlockSpec((bn, h0), lambda i: (i, 0)),
            pl.BlockSpec((bn, h0), lambda i: (i, 0)),
            pl.BlockSpec((h1, h0), full),
            pl.BlockSpec((1, h1), full),
            pl.BlockSpec((h1, h0), full),
        ],
        out_specs=pl.BlockSpec((bn, h1), lambda i: (i, 0)),
        out_shape=jax.ShapeDtypeStruct((n, h1), jnp.float32),
    )(part2, degb, h, W_l1, b_l1.reshape(1, h1), W_r1)

    return out
